# baseline (speedup 1.0000x reference)
import jax
import jax.numpy as jnp
from jax.experimental import pallas as pl

_K = 32
_ROWS = 256
_LANES = 128


def _f16_round_bits(x):
    b = jax.lax.bitcast_convert_type(x, jnp.int32)
    neg = b < 0
    mag = b & 0x7FFFFFFF
    rounded = (mag + 0xFFF + ((mag >> 13) & 1)) & ~0x1FFF
    return rounded, neg


def _mask_kernel(v_ref, a_ref, p_ref, o_ref):
    a = a_ref[0]
    p = p_ref[0]
    acc = jax.lax.dot_general(
        a, p, (((1,), (1,)), ((), ())), preferred_element_type=jnp.float32)
    pm_bits, pm_neg = _f16_round_bits(acc)
    pm = jax.lax.bitcast_convert_type(
        jnp.where(pm_neg, pm_bits | jnp.int32(-0x80000000), pm_bits), jnp.float32)
    prod = v_ref[0] * pm
    prod = jnp.where(jnp.isnan(prod), 0.0, prod)

    pr, neg = _f16_round_bits(prod)
    mag = jnp.clip((pr >> 13) - (112 << 10), 0, 32767)
    skey32 = jnp.where(neg, -1 - mag, mag)
    skey = skey32.astype(jnp.int16)
    r, s = prod.shape
    rev32 = (s - 1) - jax.lax.broadcasted_iota(jnp.int32, (r, s), 1)
    rev_idx = rev32.astype(jnp.int16)

    def count_ge(arr16, cand32):
        c16 = cand32.astype(jnp.int16)
        return jnp.sum((arr16 >= c16).astype(jnp.int16), axis=1,
                       keepdims=True).astype(jnp.int32)

    cnt0 = count_ge(skey, jnp.zeros((r, 1), jnp.int32))
    t0 = jnp.where(cnt0 >= _K, 0, -32768)

    def vstep(i, t):
        cand = t | jnp.right_shift(jnp.int32(1 << 14), i)
        return jnp.where(count_ge(skey, cand) >= _K, cand, t)

    tv = jax.lax.fori_loop(0, 15, vstep, t0)

    cnt_gt = jnp.where(tv >= 32767, 0, count_ge(skey, tv + 1))
    m = _K - cnt_gt
    ikey = jnp.where(skey == tv.astype(jnp.int16), rev_idx, jnp.int16(-1))

    def istep(i, t):
        cand = t | jnp.right_shift(jnp.int32(1 << 10), i)
        return jnp.where(count_ge(ikey, cand) >= m, cand, t)

    ti = jax.lax.fori_loop(0, 11, istep, jnp.zeros((r, 1), jnp.int32))

    keep = (skey32 > tv) | ((skey32 == tv) & (rev32 >= ti))
    o_ref[0] = jnp.where(keep, 0.0, -10000.0).astype(jnp.float32)


def kernel(qk, bucket_size):
    del bucket_size
    qk = jax.lax.stop_gradient(qk).astype(jnp.float16)
    batch, heads, seq, dim = qk.shape
    qk_norm = jnp.linalg.norm(qk, axis=-1, keepdims=True)
    phi = jnp.max(qk_norm)
    qk_const = jnp.sqrt(jnp.power(phi, 2) - jnp.power(qk_norm, 2))
    q = jnp.concatenate((qk, jnp.zeros(qk_const.shape, dtype=jnp.float16)), axis=-1)
    p = jnp.concatenate((qk, qk_const), axis=-1)
    p_norm = jnp.linalg.norm(p, axis=-1, keepdims=True)
    q_norm = jnp.linalg.norm(q, axis=-1, keepdims=True)
    m = jnp.max(p_norm)
    p = p / p_norm * m
    q = q / q_norm * m
    a = jax.random.normal(jax.random.key(1), (batch, heads, seq, dim + 1),
                          dtype=jnp.float32).astype(jnp.float16)
    v = jnp.sum(q * a, axis=-1)

    bh = batch * heads
    pad = _LANES - (dim + 1)
    a_pad = jnp.pad(a, ((0, 0), (0, 0), (0, 0), (0, pad))).reshape(
        bh, seq, _LANES).astype(jnp.float32)
    p_pad = jnp.pad(p, ((0, 0), (0, 0), (0, 0), (0, pad))).reshape(
        bh, seq, _LANES).astype(jnp.float32)
    v3 = v.reshape(bh, 1, seq).astype(jnp.float32)

    out = pl.pallas_call(
        _mask_kernel,
        grid=(bh, seq // _ROWS),
        in_specs=[
            pl.BlockSpec((1, 1, seq), lambda h, r: (h, 0, 0)),
            pl.BlockSpec((1, _ROWS, _LANES), lambda h, r: (h, r, 0)),
            pl.BlockSpec((1, seq, _LANES), lambda h, r: (h, 0, 0)),
        ],
        out_specs=pl.BlockSpec((1, _ROWS, seq), lambda h, r: (h, r, 0)),
        out_shape=jax.ShapeDtypeStruct((bh, seq, seq), jnp.float32),
    )(v3, a_pad, p_pad)
    return out.reshape(batch, heads, seq, seq)


# speedup vs baseline: 1.4853x; 1.4853x over previous
import jax
import jax.numpy as jnp
from jax.experimental import pallas as pl
from jax.experimental.pallas import tpu as pltpu

_K = 32
_ROWS = 256
_LANES = 128


def _f16_round_bits(x):
    b = jax.lax.bitcast_convert_type(x, jnp.int32)
    neg = b < 0
    mag = b & 0x7FFFFFFF
    rounded = (mag + 0xFFF + ((mag >> 13) & 1)) & ~0x1FFF
    return rounded, neg


def _mask_kernel(v_ref, a_ref, p_ref, o_ref):
    a = a_ref[0]
    p = p_ref[0]
    acc = jax.lax.dot_general(
        a, p, (((1,), (1,)), ((), ())), preferred_element_type=jnp.float32)
    pm_bits, pm_neg = _f16_round_bits(acc)
    pm = jax.lax.bitcast_convert_type(
        jnp.where(pm_neg, pm_bits | jnp.int32(-0x80000000), pm_bits), jnp.float32)
    prod = v_ref[0] * pm
    prod = jnp.where(jnp.isnan(prod), 0.0, prod)

    pr, neg = _f16_round_bits(prod)
    val = jnp.where(neg, -(pr >> 13), pr >> 13) + (1 << 18)
    r, s = prod.shape
    rev_idx = (s - 1) - jax.lax.broadcasted_iota(jnp.int32, (r, s), 1)
    keys = (val << 11) | rev_idx

    def bit_step(i, t):
        cand = t | (1 << (29 - i))
        cnt = jnp.sum((keys >= cand).astype(jnp.int32), axis=1, keepdims=True)
        return jnp.where(cnt >= _K, cand, t)

    thresh = jax.lax.fori_loop(0, 30, bit_step, jnp.zeros((r, 1), jnp.int32))
    o_ref[0] = jnp.where(keys >= thresh, 0.0, -10000.0).astype(jnp.float32)


def kernel(qk, bucket_size):
    del bucket_size
    qk = jax.lax.stop_gradient(qk).astype(jnp.float16)
    batch, heads, seq, dim = qk.shape
    qk_norm = jnp.linalg.norm(qk, axis=-1, keepdims=True)
    phi = jnp.max(qk_norm)
    qk_const = jnp.sqrt(jnp.power(phi, 2) - jnp.power(qk_norm, 2))
    q = jnp.concatenate((qk, jnp.zeros(qk_const.shape, dtype=jnp.float16)), axis=-1)
    p = jnp.concatenate((qk, qk_const), axis=-1)
    p_norm = jnp.linalg.norm(p, axis=-1, keepdims=True)
    q_norm = jnp.linalg.norm(q, axis=-1, keepdims=True)
    m = jnp.max(p_norm)
    p = p / p_norm * m
    q = q / q_norm * m
    a = jax.random.normal(jax.random.key(1), (batch, heads, seq, dim + 1),
                          dtype=jnp.float32).astype(jnp.float16)
    v = jnp.sum(q * a, axis=-1)

    bh = batch * heads
    pad = _LANES - (dim + 1)
    a_pad = jnp.pad(a, ((0, 0), (0, 0), (0, 0), (0, pad))).reshape(
        bh, seq, _LANES).astype(jnp.float32)
    p_pad = jnp.pad(p, ((0, 0), (0, 0), (0, 0), (0, pad))).reshape(
        bh, seq, _LANES).astype(jnp.float32)
    v3 = v.reshape(bh, 1, seq).astype(jnp.float32)

    out = pl.pallas_call(
        _mask_kernel,
        grid=(bh, seq // _ROWS),
        in_specs=[
            pl.BlockSpec((1, 1, seq), lambda h, r: (h, 0, 0)),
            pl.BlockSpec((1, _ROWS, _LANES), lambda h, r: (h, r, 0)),
            pl.BlockSpec((1, seq, _LANES), lambda h, r: (h, 0, 0)),
        ],
        out_specs=pl.BlockSpec((1, _ROWS, seq), lambda h, r: (h, r, 0)),
        out_shape=jax.ShapeDtypeStruct((bh, seq, seq), jnp.float32),
        compiler_params=pltpu.CompilerParams(
            dimension_semantics=("parallel", "parallel")),
    )(v3, a_pad, p_pad)
    return out.reshape(batch, heads, seq, seq)


# speedup vs baseline: 1.7929x; 1.2071x over previous
import jax
import jax.numpy as jnp
from jax.experimental import pallas as pl
from jax.experimental.pallas import tpu as pltpu

_K = 32
_ROWS = 256
_LANES = 128


def _f16_round_bits(x):
    b = jax.lax.bitcast_convert_type(x, jnp.int32)
    neg = b < 0
    mag = b & 0x7FFFFFFF
    rounded = (mag + 0xFFF + ((mag >> 13) & 1)) & ~0x1FFF
    return rounded, neg


def _mask_kernel(v_ref, a_ref, p_ref, o_ref):
    a = a_ref[0]
    p = p_ref[0]
    acc = jax.lax.dot_general(
        a, p, (((1,), (1,)), ((), ())), preferred_element_type=jnp.float32)
    pm_bits, pm_neg = _f16_round_bits(acc)
    pm = jax.lax.bitcast_convert_type(
        jnp.where(pm_neg, pm_bits | jnp.int32(-0x80000000), pm_bits), jnp.float32)
    prod = v_ref[0] * pm
    prod = jnp.where(jnp.isnan(prod), 0.0, prod)

    pr, neg = _f16_round_bits(prod)
    val = jnp.where(neg, -(pr >> 13), pr >> 13) + (1 << 18)
    r, s = prod.shape
    rev_idx = (s - 1) - jax.lax.broadcasted_iota(jnp.int32, (r, s), 1)
    keys = (val << 11) | rev_idx

    thresh = jnp.zeros((r, 1), jnp.int32)
    for bit in range(29, -1, -1):
        cand = thresh | (1 << bit)
        cnt = jnp.sum((keys >= cand).astype(jnp.int32), axis=1, keepdims=True)
        thresh = jnp.where(cnt >= _K, cand, thresh)
    o_ref[0] = jnp.where(keys >= thresh, 0.0, -10000.0).astype(jnp.float32)


def kernel(qk, bucket_size):
    del bucket_size
    qk = jax.lax.stop_gradient(qk).astype(jnp.float16)
    batch, heads, seq, dim = qk.shape
    qk_norm = jnp.linalg.norm(qk, axis=-1, keepdims=True)
    phi = jnp.max(qk_norm)
    qk_const = jnp.sqrt(jnp.power(phi, 2) - jnp.power(qk_norm, 2))
    q = jnp.concatenate((qk, jnp.zeros(qk_const.shape, dtype=jnp.float16)), axis=-1)
    p = jnp.concatenate((qk, qk_const), axis=-1)
    p_norm = jnp.linalg.norm(p, axis=-1, keepdims=True)
    q_norm = jnp.linalg.norm(q, axis=-1, keepdims=True)
    m = jnp.max(p_norm)
    p = p / p_norm * m
    q = q / q_norm * m
    a = jax.random.normal(jax.random.key(1), (batch, heads, seq, dim + 1),
                          dtype=jnp.float32).astype(jnp.float16)
    v = jnp.sum(q * a, axis=-1)

    bh = batch * heads
    pad = _LANES - (dim + 1)
    a_pad = jnp.pad(a, ((0, 0), (0, 0), (0, 0), (0, pad))).reshape(
        bh, seq, _LANES).astype(jnp.float32)
    p_pad = jnp.pad(p, ((0, 0), (0, 0), (0, 0), (0, pad))).reshape(
        bh, seq, _LANES).astype(jnp.float32)
    v3 = v.reshape(bh, 1, seq).astype(jnp.float32)

    out = pl.pallas_call(
        _mask_kernel,
        grid=(bh, seq // _ROWS),
        in_specs=[
            pl.BlockSpec((1, 1, seq), lambda h, r: (h, 0, 0)),
            pl.BlockSpec((1, _ROWS, _LANES), lambda h, r: (h, r, 0)),
            pl.BlockSpec((1, seq, _LANES), lambda h, r: (h, 0, 0)),
        ],
        out_specs=pl.BlockSpec((1, _ROWS, seq), lambda h, r: (h, r, 0)),
        out_shape=jax.ShapeDtypeStruct((bh, seq, seq), jnp.float32),
        compiler_params=pltpu.CompilerParams(
            dimension_semantics=("parallel", "parallel")),
    )(v3, a_pad, p_pad)
    return out.reshape(batch, heads, seq, seq)


# speedup vs baseline: 1.8145x; 1.0120x over previous
import jax
import jax.numpy as jnp
from jax.experimental import pallas as pl
from jax.experimental.pallas import tpu as pltpu

_K = 32
_ROWS = 256
_LANES = 128


def _f16_round_bits(x):
    b = jax.lax.bitcast_convert_type(x, jnp.int32)
    neg = b < 0
    mag = b & 0x7FFFFFFF
    rounded = (mag + 0xFFF + ((mag >> 13) & 1)) & ~0x1FFF
    return rounded, neg


def _mask_kernel(v_ref, a_ref, p_ref, o_ref):
    a = a_ref[0]
    p = p_ref[0]
    acc = jax.lax.dot_general(
        a, p, (((1,), (1,)), ((), ())), preferred_element_type=jnp.float32)
    pm_bits, pm_neg = _f16_round_bits(acc)
    pm = jax.lax.bitcast_convert_type(
        jnp.where(pm_neg, pm_bits | jnp.int32(-0x80000000), pm_bits), jnp.float32)
    prod = v_ref[0] * pm

    pr, neg = _f16_round_bits(prod)
    val = jnp.where(neg, -(pr >> 13), pr >> 13) + (1 << 18)
    r, s = prod.shape
    rev_idx = (s - 1) - jax.lax.broadcasted_iota(jnp.int32, (r, s), 1)
    keys = (val << 11) | rev_idx

    thresh = jnp.zeros((r, 1), jnp.int32)
    for bit in range(29, -1, -1):
        cand = thresh | (1 << bit)
        cnt = jnp.sum((keys >= cand).astype(jnp.int32), axis=1, keepdims=True)
        thresh = jnp.where(cnt >= _K, cand, thresh)
    o_ref[0] = jnp.where(keys >= thresh, 0.0, -10000.0).astype(jnp.float32)


def kernel(qk, bucket_size):
    del bucket_size
    qk = jax.lax.stop_gradient(qk).astype(jnp.float16)
    batch, heads, seq, dim = qk.shape
    qk_norm = jnp.linalg.norm(qk, axis=-1, keepdims=True)
    phi = jnp.max(qk_norm)
    qk_const = jnp.sqrt(jnp.power(phi, 2) - jnp.power(qk_norm, 2))
    q = jnp.concatenate((qk, jnp.zeros(qk_const.shape, dtype=jnp.float16)), axis=-1)
    p = jnp.concatenate((qk, qk_const), axis=-1)
    p_norm = jnp.linalg.norm(p, axis=-1, keepdims=True)
    q_norm = jnp.linalg.norm(q, axis=-1, keepdims=True)
    m = jnp.max(p_norm)
    p = p / p_norm * m
    q = q / q_norm * m
    a = jax.random.normal(jax.random.key(1), (batch, heads, seq, dim + 1),
                          dtype=jnp.float32).astype(jnp.float16)
    v = jnp.sum(q * a, axis=-1)

    bh = batch * heads
    pad = _LANES - (dim + 1)
    a_pad = jnp.pad(a, ((0, 0), (0, 0), (0, 0), (0, pad))).reshape(
        bh, seq, _LANES).astype(jnp.float32)
    p_pad = jnp.pad(p, ((0, 0), (0, 0), (0, 0), (0, pad))).reshape(
        bh, seq, _LANES).astype(jnp.float32)
    v3 = v.reshape(bh, 1, seq).astype(jnp.float32)

    out = pl.pallas_call(
        _mask_kernel,
        grid=(bh, seq // _ROWS),
        in_specs=[
            pl.BlockSpec((1, 1, seq), lambda h, r: (h, 0, 0)),
            pl.BlockSpec((1, _ROWS, _LANES), lambda h, r: (h, r, 0)),
            pl.BlockSpec((1, seq, _LANES), lambda h, r: (h, 0, 0)),
        ],
        out_specs=pl.BlockSpec((1, _ROWS, seq), lambda h, r: (h, r, 0)),
        out_shape=jax.ShapeDtypeStruct((bh, seq, seq), jnp.float32),
        compiler_params=pltpu.CompilerParams(
            dimension_semantics=("parallel", "parallel")),
    )(v3, a_pad, p_pad)
    return out.reshape(batch, heads, seq, seq)


# speedup vs baseline: 1.8345x; 1.0111x over previous
import jax
import jax.numpy as jnp
from jax.experimental import pallas as pl
from jax.experimental.pallas import tpu as pltpu

_K = 32
_ROWS = 512
_LANES = 128


def _f16_round_bits(x):
    b = jax.lax.bitcast_convert_type(x, jnp.int32)
    neg = b < 0
    mag = b & 0x7FFFFFFF
    rounded = (mag + 0xFFF + ((mag >> 13) & 1)) & ~0x1FFF
    return rounded, neg


def _mask_kernel(v_ref, a_ref, p_ref, o_ref):
    a = a_ref[0]
    p = p_ref[0]
    acc = jax.lax.dot_general(
        a, p, (((1,), (1,)), ((), ())), preferred_element_type=jnp.float32)
    pm_bits, pm_neg = _f16_round_bits(acc)
    pm = jax.lax.bitcast_convert_type(
        jnp.where(pm_neg, pm_bits | jnp.int32(-0x80000000), pm_bits), jnp.float32)
    prod = v_ref[0] * pm

    pr, neg = _f16_round_bits(prod)
    val = jnp.where(neg, -(pr >> 13), pr >> 13) + (1 << 18)
    r, s = prod.shape
    rev_idx = (s - 1) - jax.lax.broadcasted_iota(jnp.int32, (r, s), 1)
    keys = (val << 11) | rev_idx

    thresh = jnp.zeros((r, 1), jnp.int32)
    for bit in range(29, -1, -1):
        cand = thresh | (1 << bit)
        cnt = jnp.sum((keys >= cand).astype(jnp.int32), axis=1, keepdims=True)
        thresh = jnp.where(cnt >= _K, cand, thresh)
    o_ref[0] = jnp.where(keys >= thresh, 0.0, -10000.0).astype(jnp.float32)


def kernel(qk, bucket_size):
    del bucket_size
    qk = jax.lax.stop_gradient(qk).astype(jnp.float16)
    batch, heads, seq, dim = qk.shape
    qk_norm = jnp.linalg.norm(qk, axis=-1, keepdims=True)
    phi = jnp.max(qk_norm)
    qk_const = jnp.sqrt(jnp.power(phi, 2) - jnp.power(qk_norm, 2))
    q = jnp.concatenate((qk, jnp.zeros(qk_const.shape, dtype=jnp.float16)), axis=-1)
    p = jnp.concatenate((qk, qk_const), axis=-1)
    p_norm = jnp.linalg.norm(p, axis=-1, keepdims=True)
    q_norm = jnp.linalg.norm(q, axis=-1, keepdims=True)
    m = jnp.max(p_norm)
    p = p / p_norm * m
    q = q / q_norm * m
    a = jax.random.normal(jax.random.key(1), (batch, heads, seq, dim + 1),
                          dtype=jnp.float32).astype(jnp.float16)
    v = jnp.sum(q * a, axis=-1)

    bh = batch * heads
    pad = _LANES - (dim + 1)
    a_pad = jnp.pad(a, ((0, 0), (0, 0), (0, 0), (0, pad))).reshape(
        bh, seq, _LANES).astype(jnp.float32)
    p_pad = jnp.pad(p, ((0, 0), (0, 0), (0, 0), (0, pad))).reshape(
        bh, seq, _LANES).astype(jnp.float32)
    v3 = v.reshape(bh, 1, seq).astype(jnp.float32)

    out = pl.pallas_call(
        _mask_kernel,
        grid=(bh, seq // _ROWS),
        in_specs=[
            pl.BlockSpec((1, 1, seq), lambda h, r: (h, 0, 0)),
            pl.BlockSpec((1, _ROWS, _LANES), lambda h, r: (h, r, 0)),
            pl.BlockSpec((1, seq, _LANES), lambda h, r: (h, 0, 0)),
        ],
        out_specs=pl.BlockSpec((1, _ROWS, seq), lambda h, r: (h, r, 0)),
        out_shape=jax.ShapeDtypeStruct((bh, seq, seq), jnp.float32),
        compiler_params=pltpu.CompilerParams(
            dimension_semantics=("parallel", "parallel")),
    )(v3, a_pad, p_pad)
    return out.reshape(batch, heads, seq, seq)
